# baseline (speedup 1.0000x reference)
import jax
import jax.numpy as jnp
from jax.experimental import pallas as pl
from jax.experimental.pallas import tpu as pltpu


def _moe_body(scale_ref, x_ref, w_ref, b_ref, o_ref):
    x = x_ref[...].astype(jnp.bfloat16)
    w = w_ref[0].astype(jnp.bfloat16)
    acc = jax.lax.dot_general(
        x, w, (((1,), (1,)), ((), ())), preferred_element_type=jnp.float32
    )
    o_ref[...] = (acc + b_ref[0]) * scale_ref[0, 0]


def kernel(x, W, b, temperature):
    n, d = x.shape
    e = W.shape[0]
    per = n // e
    tm = 2048
    scale = jnp.exp(
        jnp.minimum(temperature, jnp.log(jnp.float32(100.0)))
    ).reshape(1, 1)

    out = pl.pallas_call(
        _moe_body,
        grid=(e, per // tm),
        in_specs=[
            pl.BlockSpec(memory_space=pltpu.SMEM),
            pl.BlockSpec((tm, d), lambda ei, ti: (ei * (per // tm) + ti, 0)),
            pl.BlockSpec((1, d, d), lambda ei, ti: (ei, 0, 0)),
            pl.BlockSpec((1, 1, d), lambda ei, ti: (ei, 0, 0)),
        ],
        out_specs=pl.BlockSpec((tm, d), lambda ei, ti: (ei * (per // tm) + ti, 0)),
        out_shape=jax.ShapeDtypeStruct((n, d), x.dtype),
        compiler_params=pltpu.CompilerParams(
            dimension_semantics=("arbitrary", "arbitrary"),
        ),
    )(scale, x, W, b.reshape(e, 1, d))

    aux_loss = jnp.float32(0.0)
    return (out, aux_loss)


# speedup vs baseline: 1.0753x; 1.0753x over previous
import jax
import jax.numpy as jnp
from jax.experimental import pallas as pl
from jax.experimental.pallas import tpu as pltpu


def _moe_body(scale_ref, x_ref, w_ref, b_ref, o_ref):
    x = x_ref[...]
    w = w_ref[0]
    acc = jax.lax.dot_general(
        x, w, (((1,), (1,)), ((), ())), preferred_element_type=jnp.float32
    )
    o_ref[...] = (acc + b_ref[0]) * scale_ref[0, 0]


def kernel(x, W, b, temperature):
    n, d = x.shape
    e = W.shape[0]
    per = n // e
    tm = 4096
    scale = jnp.exp(
        jnp.minimum(temperature, jnp.log(jnp.float32(100.0)))
    ).reshape(1, 1)

    out = pl.pallas_call(
        _moe_body,
        grid=(e, per // tm),
        in_specs=[
            pl.BlockSpec(memory_space=pltpu.SMEM),
            pl.BlockSpec((tm, d), lambda ei, ti: (ei * (per // tm) + ti, 0)),
            pl.BlockSpec((1, d, d), lambda ei, ti: (ei, 0, 0)),
            pl.BlockSpec((1, 1, d), lambda ei, ti: (ei, 0, 0)),
        ],
        out_specs=pl.BlockSpec((tm, d), lambda ei, ti: (ei * (per // tm) + ti, 0)),
        out_shape=jax.ShapeDtypeStruct((n, d), x.dtype),
        compiler_params=pltpu.CompilerParams(
            dimension_semantics=("arbitrary", "arbitrary"),
            vmem_limit_bytes=100 * 1024 * 1024,
        ),
    )(scale, x, W, b.reshape(e, 1, d))

    aux_loss = jnp.float32(0.0)
    return (out, aux_loss)


# speedup vs baseline: 1.0763x; 1.0009x over previous
import jax
import jax.numpy as jnp
from jax.experimental import pallas as pl
from jax.experimental.pallas import tpu as pltpu


def _moe_body(scale_ref, x_ref, w_ref, b_ref, o_ref):
    x = x_ref[...]
    w = w_ref[0]
    acc = jax.lax.dot_general(
        x, w, (((1,), (1,)), ((), ())), preferred_element_type=jnp.float32
    )
    o_ref[...] = (acc + b_ref[0]) * scale_ref[0, 0]


def kernel(x, W, b, temperature):
    n, d = x.shape
    e = W.shape[0]
    per = n // e
    tm = 4096
    scale = jnp.exp(
        jnp.minimum(temperature, jnp.log(jnp.float32(100.0)))
    ).reshape(1, 1)

    out = pl.pallas_call(
        _moe_body,
        grid=(n // tm,),
        in_specs=[
            pl.BlockSpec(memory_space=pltpu.SMEM),
            pl.BlockSpec((tm, d), lambda i: (i, 0)),
            pl.BlockSpec((1, d, d), lambda i: (i * tm // per, 0, 0)),
            pl.BlockSpec((1, 1, d), lambda i: (i * tm // per, 0, 0)),
        ],
        out_specs=pl.BlockSpec((tm, d), lambda i: (i, 0)),
        out_shape=jax.ShapeDtypeStruct((n, d), x.dtype),
        compiler_params=pltpu.CompilerParams(
            dimension_semantics=("arbitrary",),
            vmem_limit_bytes=100 * 1024 * 1024,
        ),
    )(scale, x, W, b.reshape(e, 1, d))

    aux_loss = jnp.float32(0.0)
    return (out, aux_loss)


# speedup vs baseline: 1.0773x; 1.0010x over previous
import jax
import jax.numpy as jnp
from jax.experimental import pallas as pl
from jax.experimental.pallas import tpu as pltpu


def _moe_body(temp_ref, x_ref, w_ref, b_ref, o_ref):
    x = x_ref[...]
    w = w_ref[0]
    acc = jax.lax.dot_general(
        x, w, (((1,), (1,)), ((), ())), preferred_element_type=jnp.float32
    )
    scale = jnp.exp(jnp.minimum(temp_ref[0, 0], jnp.log(jnp.float32(100.0))))
    o_ref[...] = (acc + b_ref[0]) * scale


def kernel(x, W, b, temperature):
    n, d = x.shape
    e = W.shape[0]
    per = n // e
    tm = 4096
    temp2d = temperature.reshape(1, 1)

    out = pl.pallas_call(
        _moe_body,
        grid=(n // tm,),
        in_specs=[
            pl.BlockSpec(memory_space=pltpu.SMEM),
            pl.BlockSpec((tm, d), lambda i: (i, 0)),
            pl.BlockSpec((1, d, d), lambda i: (i * tm // per, 0, 0)),
            pl.BlockSpec((1, 1, d), lambda i: (i * tm // per, 0, 0)),
        ],
        out_specs=pl.BlockSpec((tm, d), lambda i: (i, 0)),
        out_shape=jax.ShapeDtypeStruct((n, d), x.dtype),
        compiler_params=pltpu.CompilerParams(
            dimension_semantics=("arbitrary",),
            vmem_limit_bytes=100 * 1024 * 1024,
        ),
    )(temp2d, x, W, b.reshape(e, 1, d))

    aux_loss = jnp.float32(0.0)
    return (out, aux_loss)
